# Initial kernel scaffold; baseline (speedup 1.0000x reference)
import jax
import jax.numpy as jnp
from jax.experimental import pallas as pl


def kernel(w1, noise_ch, nphi, vx, vy, dv1, s1, bnd_idx, bnd_idy):
    raise NotImplementedError("write your pallas kernel here")


# speedup vs baseline: 1.5145x; 1.5145x over previous
import functools

import jax
import jax.numpy as jnp
from jax import lax
from jax.experimental import pallas as pl
from jax.experimental.pallas import tpu as pltpu
from jax.experimental.pallas import tpu_sc as plsc

PR, PC = 5, 5
NR, NC = 128, 128
NB = 40
ALPHA = 15.0
Q = PR * PC
L = 16
NWORK = 32
IPW = NR // NWORK
PASSES = 2
CPP = 2 * NC
RPP = 2 * PR
NBP = 48


def _splat(v):
    return jnp.full((L,), v, jnp.int32)


def _mgmc_body(w1_h, ns_h, np_h, vx_h, vy_h, dv_h, s1_h,
               ix0_h, ix1_h, iy0_h, iy1_h, out_h,
               w1b, dvb, u1, nsb, npb, vxb, vyb, uxb, uyb,
               s1b, accb, cb, ix0b, ix1b, iy0b, iy1b):
    wid = lax.axis_index("s") * 2 + lax.axis_index("c")
    iota = lax.iota(jnp.int32, L)

    pltpu.sync_copy(ix0_h, ix0b)
    pltpu.sync_copy(ix1_h, ix1b)
    pltpu.sync_copy(iy0_h, iy0b)
    pltpu.sync_copy(iy1_h, iy1b)

    for p in range(PASSES):
        ii0 = wid * IPW + p * 2
        r0 = ii0 * PR
        c0 = ii0 * NC

        pltpu.sync_copy(w1_h.at[pl.ds(r0, RPP)], w1b)
        pltpu.sync_copy(dv_h.at[pl.ds(r0, RPP)], dvb)
        pltpu.sync_copy(ns_h.at[:, pl.ds(c0, CPP)], nsb)
        pltpu.sync_copy(np_h.at[:, pl.ds(c0, CPP)], npb)
        pltpu.sync_copy(vx_h.at[:, pl.ds(c0, CPP)], vxb)
        pltpu.sync_copy(vy_h.at[:, pl.ds(c0, CPP)], vyb)
        pltpu.sync_copy(s1_h.at[pl.ds(c0, CPP)], s1b)

        def build_ch(ch, _):
            jjv = ch * L + iota
            for q in range(Q):
                for il in range(2):
                    rowv = _splat(il * PR + q % PR)
                    colv = jjv * PC + (q // PR)
                    val = plsc.load_gather(w1b, [rowv, colv])
                    u1[q, pl.ds(il * NC + ch * L, L)] = val
            return 0
        lax.fori_loop(0, NC // L, build_ch, 0, unroll=False)

        def acc_ch(ch, _):
            sl = pl.ds(ch * L, L)
            a = jnp.zeros((L,), jnp.float32)
            for q in range(Q):
                a = a + (nsb[q, sl] - u1[q, sl]) * npb[q, sl]
            accb[sl] = a
            cb[sl] = jnp.zeros((L,), jnp.float32)
            return 0
        lax.fori_loop(0, CPP // L, acc_ch, 0, unroll=False)

        def bld_row(b, _):
            rx0 = plsc.load_gather(ix0b, [_splat(b)])
            rx1 = plsc.load_gather(ix1b, [_splat(b)])
            ry0 = plsc.load_gather(iy0b, [_splat(b)])
            ry1 = plsc.load_gather(iy1b, [_splat(b)])
            def bld_ch(ch, _):
                colv = ch * L + iota
                sl = pl.ds(ch * L, L)
                uxb[b, sl] = (plsc.load_gather(u1, [rx0, colv])
                              - plsc.load_gather(u1, [rx1, colv]))
                uyb[b, sl] = (plsc.load_gather(u1, [ry0, colv])
                              - plsc.load_gather(u1, [ry1, colv]))
                return 0
            lax.fori_loop(0, CPP // L, bld_ch, 0, unroll=False)
            return 0
        lax.fori_loop(0, NB, bld_row, 0, unroll=False)

        def fp(it, _):
            def ch_body(ch, _):
                sl = pl.ds(ch * L, L)
                cv = cb[sl]
                def row(b, carry):
                    f, s = carry
                    uxv = uxb[b, sl]
                    uyv = uyb[b, sl]
                    vxv = vxb[b, sl]
                    vyv = vyb[b, sl]
                    ucx = uxv + cv * vxv
                    ucy = uyv + cv * vyv
                    u2 = ucx * ucx + ucy * ucy + 1e-4
                    ib = plsc.bitcast(u2, jnp.int32)
                    y = plsc.bitcast(jnp.int32(0x5F3759DF) - (ib >> 1),
                                     jnp.float32)
                    y = y * (1.5 - (0.5 * u2) * y * y)
                    f = f + (vxv * vxv + vyv * vyv) * y
                    s = s + (uxv * vxv + uyv * vyv) * y
                    return f, s
                z = jnp.zeros((L,), jnp.float32)
                f, s = lax.fori_loop(0, NB, row, (z, z), unroll=False)
                cb[sl] = (accb[sl] - ALPHA * s) / (ALPHA * f + s1b[sl])
                return 0
            lax.fori_loop(0, CPP // L, ch_body, 0, unroll=False)
            return 0
        lax.fori_loop(0, 6, fp, 0, unroll=False)

        for rl in range(RPP):
            il = rl // PR
            def out_ch(ch, _):
                colv = ch * L + iota
                sl = pl.ds(ch * L, L)
                cg = plsc.load_gather(cb, [il * NC + colv // PC])
                w1b[rl, sl] = w1b[rl, sl] + cg * dvb[rl, sl]
                return 0
            lax.fori_loop(0, (NC * PC) // L, out_ch, 0, unroll=False)

        pltpu.sync_copy(w1b, out_h.at[pl.ds(r0, RPP)])


def kernel(w1, noise_ch, nphi, vx, vy, dv1, s1, bnd_idx, bnd_idy):
    def pad48(col):
        return jnp.zeros((NBP,), jnp.int32).at[:NB].set(col.astype(jnp.int32))

    ix0, ix1 = pad48(bnd_idx[:, 0]), pad48(bnd_idx[:, 1])
    iy0, iy1 = pad48(bnd_idy[:, 0]), pad48(bnd_idy[:, 1])

    mesh = plsc.VectorSubcoreMesh(core_axis_name="c", subcore_axis_name="s",
                                  num_cores=2, num_subcores=16)
    run = pl.kernel(
        _mgmc_body,
        out_type=jax.ShapeDtypeStruct((NR * PR, NC * PC), jnp.float32),
        mesh=mesh,
        compiler_params=pltpu.CompilerParams(use_tc_tiling_on_sc=False,
                                             needs_layout_passes=False),
        scratch_types=[
            pltpu.VMEM((RPP, NC * PC), jnp.float32),
            pltpu.VMEM((RPP, NC * PC), jnp.float32),
            pltpu.VMEM((Q, CPP), jnp.float32),
            pltpu.VMEM((Q, CPP), jnp.float32),
            pltpu.VMEM((Q, CPP), jnp.float32),
            pltpu.VMEM((NB, CPP), jnp.float32),
            pltpu.VMEM((NB, CPP), jnp.float32),
            pltpu.VMEM((NB, CPP), jnp.float32),
            pltpu.VMEM((NB, CPP), jnp.float32),
            pltpu.VMEM((CPP,), jnp.float32),
            pltpu.VMEM((CPP,), jnp.float32),
            pltpu.VMEM((CPP,), jnp.float32),
            pltpu.VMEM((NBP,), jnp.int32),
            pltpu.VMEM((NBP,), jnp.int32),
            pltpu.VMEM((NBP,), jnp.int32),
            pltpu.VMEM((NBP,), jnp.int32),
        ],
    )
    return run(w1, noise_ch, nphi, vx, vy, dv1, s1, ix0, ix1, iy0, iy1)


# speedup vs baseline: 1.8196x; 1.2015x over previous
import functools

import jax
import jax.numpy as jnp
from jax import lax
from jax.experimental import pallas as pl
from jax.experimental.pallas import tpu as pltpu
from jax.experimental.pallas import tpu_sc as plsc

PR, PC = 5, 5
NR, NC = 128, 128
NB = 40
ALPHA = 15.0
Q = PR * PC
L = 16
NWORK = 32
IPW = NR // NWORK
PASSES = 2
CPP = 2 * NC
RPP = 2 * PR
NBP = 48


def _splat(v):
    return jnp.full((L,), v, jnp.int32)


def _mgmc_body(w1_h, ns_h, np_h, vx_h, vy_h, dv_h, s1_h,
               ix0_h, ix1_h, iy0_h, iy1_h, out_h,
               w1b, dvb, u1, nsb, npb, vxb, vyb, uxb, uyb,
               s1b, accb, cb, ix0b, ix1b, iy0b, iy1b, sems, osem):
    wid = lax.axis_index("s") * 2 + lax.axis_index("c")
    iota = lax.iota(jnp.int32, L)

    pltpu.sync_copy(ix0_h, ix0b)
    pltpu.sync_copy(ix1_h, ix1b)
    pltpu.sync_copy(iy0_h, iy0b)
    pltpu.sync_copy(iy1_h, iy1b)

    for p in range(PASSES):
        ii0 = wid * IPW + p * 2
        r0 = ii0 * PR
        c0 = ii0 * NC

        if p > 0:
            cp_out.wait()
        cp_w1 = pltpu.async_copy(w1_h.at[pl.ds(r0, RPP)], w1b, sems.at[0])
        cp_ns = pltpu.async_copy(ns_h.at[:, pl.ds(c0, CPP)], nsb, sems.at[1])
        cp_np = pltpu.async_copy(np_h.at[:, pl.ds(c0, CPP)], npb, sems.at[2])
        cp_vx = pltpu.async_copy(vx_h.at[:, pl.ds(c0, CPP)], vxb, sems.at[3])
        cp_vy = pltpu.async_copy(vy_h.at[:, pl.ds(c0, CPP)], vyb, sems.at[4])
        cp_s1 = pltpu.async_copy(s1_h.at[pl.ds(c0, CPP)], s1b, sems.at[5])
        cp_dv = pltpu.async_copy(dv_h.at[pl.ds(r0, RPP)], dvb, sems.at[6])

        cp_w1.wait()

        @plsc.parallel_loop(0, NC // L, 1, unroll=2)
        def build_ch(ch):
            jjv = ch * L + iota
            for q in range(Q):
                for il in range(2):
                    rowv = _splat(il * PR + q % PR)
                    colv = jjv * PC + (q // PR)
                    val = plsc.load_gather(w1b, [rowv, colv])
                    u1[q, pl.ds(il * NC + ch * L, L)] = val

        def bld_row(b, _):
            rx0 = plsc.load_gather(ix0b, [_splat(b)])
            rx1 = plsc.load_gather(ix1b, [_splat(b)])
            ry0 = plsc.load_gather(iy0b, [_splat(b)])
            ry1 = plsc.load_gather(iy1b, [_splat(b)])

            @plsc.parallel_loop(0, CPP // L, 1, unroll=2)
            def bld_ch(ch):
                colv = ch * L + iota
                sl = pl.ds(ch * L, L)
                uxb[b, sl] = (plsc.load_gather(u1, [rx0, colv])
                              - plsc.load_gather(u1, [rx1, colv]))
                uyb[b, sl] = (plsc.load_gather(u1, [ry0, colv])
                              - plsc.load_gather(u1, [ry1, colv]))
            return 0
        lax.fori_loop(0, NB, bld_row, 0, unroll=False)

        cp_ns.wait()
        cp_np.wait()

        @plsc.parallel_loop(0, CPP // L, 1, unroll=2)
        def acc_ch(ch):
            sl = pl.ds(ch * L, L)
            a = (nsb[0, sl] - u1[0, sl]) * npb[0, sl]
            for q in range(1, Q):
                a = a + (nsb[q, sl] - u1[q, sl]) * npb[q, sl]
            accb[sl] = a
            cb[sl] = jnp.zeros((L,), jnp.float32)

        cp_vx.wait()
        cp_vy.wait()
        cp_s1.wait()

        def fp(it, _):
            @plsc.parallel_loop(0, CPP // L, 1, unroll=2)
            def ch_body(ch):
                sl = pl.ds(ch * L, L)
                cv = cb[sl]
                def row(b, carry):
                    f, s = carry
                    uxv = uxb[b, sl]
                    uyv = uyb[b, sl]
                    vxv = vxb[b, sl]
                    vyv = vyb[b, sl]
                    ucx = uxv + cv * vxv
                    ucy = uyv + cv * vyv
                    u2 = ucx * ucx + ucy * ucy + 1e-4
                    ib = plsc.bitcast(u2, jnp.int32)
                    y = plsc.bitcast(jnp.int32(0x5F3759DF) - (ib >> 1),
                                     jnp.float32)
                    y = y * (1.5 - (0.5 * u2) * y * y)
                    f = f + (vxv * vxv + vyv * vyv) * y
                    s = s + (uxv * vxv + uyv * vyv) * y
                    return f, s
                z = jnp.zeros((L,), jnp.float32)
                f, s = lax.fori_loop(0, NB, row, (z, z), unroll=False)
                cb[sl] = (accb[sl] - ALPHA * s) / (ALPHA * f + s1b[sl])
            return 0
        lax.fori_loop(0, 6, fp, 0, unroll=False)

        cp_dv.wait()
        for rl in range(RPP):
            il = rl // PR

            @plsc.parallel_loop(0, (NC * PC) // L, 1, unroll=2)
            def out_ch(ch):
                colv = ch * L + iota
                sl = pl.ds(ch * L, L)
                cg = plsc.load_gather(cb, [il * NC + colv // PC])
                w1b[rl, sl] = w1b[rl, sl] + cg * dvb[rl, sl]

        cp_out = pltpu.async_copy(w1b, out_h.at[pl.ds(r0, RPP)], osem)
    cp_out.wait()


def kernel(w1, noise_ch, nphi, vx, vy, dv1, s1, bnd_idx, bnd_idy):
    def pad48(col):
        return jnp.zeros((NBP,), jnp.int32).at[:NB].set(col.astype(jnp.int32))

    ix0, ix1 = pad48(bnd_idx[:, 0]), pad48(bnd_idx[:, 1])
    iy0, iy1 = pad48(bnd_idy[:, 0]), pad48(bnd_idy[:, 1])

    mesh = plsc.VectorSubcoreMesh(core_axis_name="c", subcore_axis_name="s",
                                  num_cores=2, num_subcores=16)
    run = pl.kernel(
        _mgmc_body,
        out_type=jax.ShapeDtypeStruct((NR * PR, NC * PC), jnp.float32),
        mesh=mesh,
        compiler_params=pltpu.CompilerParams(use_tc_tiling_on_sc=False,
                                             needs_layout_passes=False),
        scratch_types=[
            pltpu.VMEM((RPP, NC * PC), jnp.float32),
            pltpu.VMEM((RPP, NC * PC), jnp.float32),
            pltpu.VMEM((Q, CPP), jnp.float32),
            pltpu.VMEM((Q, CPP), jnp.float32),
            pltpu.VMEM((Q, CPP), jnp.float32),
            pltpu.VMEM((NB, CPP), jnp.float32),
            pltpu.VMEM((NB, CPP), jnp.float32),
            pltpu.VMEM((NB, CPP), jnp.float32),
            pltpu.VMEM((NB, CPP), jnp.float32),
            pltpu.VMEM((CPP,), jnp.float32),
            pltpu.VMEM((CPP,), jnp.float32),
            pltpu.VMEM((CPP,), jnp.float32),
            pltpu.VMEM((NBP,), jnp.int32),
            pltpu.VMEM((NBP,), jnp.int32),
            pltpu.VMEM((NBP,), jnp.int32),
            pltpu.VMEM((NBP,), jnp.int32),
            pltpu.SemaphoreType.DMA((7,)),
            pltpu.SemaphoreType.DMA,
        ],
    )
    return run(w1, noise_ch, nphi, vx, vy, dv1, s1, ix0, ix1, iy0, iy1)


# speedup vs baseline: 1.8967x; 1.0423x over previous
import functools

import jax
import jax.numpy as jnp
from jax import lax
from jax.experimental import pallas as pl
from jax.experimental.pallas import tpu as pltpu
from jax.experimental.pallas import tpu_sc as plsc

PR, PC = 5, 5
NR, NC = 128, 128
NB = 40
ALPHA = 15.0
Q = PR * PC
L = 16
NWORK = 32
IPW = NR // NWORK
CPW = IPW * NC
RPW = IPW * PR
HCH = (CPW // 2) // L
NBP = 48


def _splat(v):
    return jnp.full((L,), v, jnp.int32)


def _mgmc_body(w1_h, ns_h, np_h, vx_h, vy_h, dvf_h, s1_h,
               ix0_h, ix1_h, iy0_h, iy1_h, out_h,
               w1b, u1f, nsb, npb, vxb, vyb, uxb, uyb,
               s1b, accb, cb, ix0b, ix1b, iy0b, iy1b, sems, osem):
    wid = lax.axis_index("s") * 2 + lax.axis_index("c")
    iota = lax.iota(jnp.int32, L)

    pltpu.sync_copy(ix0_h, ix0b)
    pltpu.sync_copy(ix1_h, ix1b)
    pltpu.sync_copy(iy0_h, iy0b)
    pltpu.sync_copy(iy1_h, iy1b)

    ii0 = wid * IPW
    r0 = ii0 * PR
    c0 = ii0 * NC

    cp_w1 = pltpu.async_copy(w1_h.at[pl.ds(r0, RPW)], w1b, sems.at[0])
    cp_ns = pltpu.async_copy(ns_h.at[:, pl.ds(c0, CPW // 2)], nsb, sems.at[1])
    cp_np = pltpu.async_copy(np_h.at[:, pl.ds(c0, CPW // 2)], npb, sems.at[2])
    cp_vx = pltpu.async_copy(vx_h.at[:, pl.ds(c0, CPW)], vxb, sems.at[3])
    cp_vy = pltpu.async_copy(vy_h.at[:, pl.ds(c0, CPW)], vyb, sems.at[4])
    cp_s1 = pltpu.async_copy(s1_h.at[pl.ds(c0, CPW)], s1b, sems.at[5])

    cp_w1.wait()

    @plsc.parallel_loop(0, CPW // L, 1, unroll=2)
    def build_ch(ch):
        il = ch // (NC // L)
        jjv = (ch % (NC // L)) * L + iota
        base = il * PR
        for q in range(Q):
            rowv = _splat(base + q % PR)
            colv = jjv * PC + (q // PR)
            val = plsc.load_gather(w1b, [rowv, colv])
            u1f[pl.ds(q * CPW + ch * L, L)] = val

    def bld_row(b, _):
        ox0 = plsc.load_gather(ix0b, [_splat(b)]) * CPW
        ox1 = plsc.load_gather(ix1b, [_splat(b)]) * CPW
        oy0 = plsc.load_gather(iy0b, [_splat(b)]) * CPW
        oy1 = plsc.load_gather(iy1b, [_splat(b)]) * CPW

        @plsc.parallel_loop(0, CPW // L, 1, unroll=2)
        def bld_ch(ch):
            colv = ch * L + iota
            sl = pl.ds(ch * L, L)
            uxb[b, sl] = (plsc.load_gather(u1f, [ox0 + colv])
                          - plsc.load_gather(u1f, [ox1 + colv]))
            uyb[b, sl] = (plsc.load_gather(u1f, [oy0 + colv])
                          - plsc.load_gather(u1f, [oy1 + colv]))
        return 0
    lax.fori_loop(0, NB, bld_row, 0, unroll=False)

    for h in range(2):
        if h == 0:
            cp_ns.wait()
            cp_np.wait()
        else:
            cp_ns2.wait()
            cp_np2.wait()

        @plsc.parallel_loop(0, HCH, 1, unroll=2)
        def acc_ch(ch):
            sl = pl.ds(ch * L, L)
            gsl = pl.ds(h * (CPW // 2) + ch * L, L)
            off = h * (CPW // 2)
            a = (nsb[0, sl] - u1f[pl.ds(off + ch * L, L)]) * npb[0, sl]
            for q in range(1, Q):
                a = a + ((nsb[q, sl] - u1f[pl.ds(q * CPW + off + ch * L, L)])
                         * npb[q, sl])
            accb[gsl] = a
            cb[gsl] = jnp.zeros((L,), jnp.float32)

        if h == 0:
            cp_ns2 = pltpu.async_copy(
                ns_h.at[:, pl.ds(c0 + CPW // 2, CPW // 2)], nsb, sems.at[1])
            cp_np2 = pltpu.async_copy(
                np_h.at[:, pl.ds(c0 + CPW // 2, CPW // 2)], npb, sems.at[2])

    cp_dv = pltpu.async_copy(dvf_h.at[pl.ds(r0 * NC * PC, RPW * NC * PC)],
                             u1f, sems.at[6])

    cp_vx.wait()
    cp_vy.wait()
    cp_s1.wait()

    def fp(it, _):
        @plsc.parallel_loop(0, CPW // L, 1, unroll=2)
        def ch_body(ch):
            sl = pl.ds(ch * L, L)
            cv = cb[sl]
            def row(b, carry):
                f, s = carry
                uxv = uxb[b, sl]
                uyv = uyb[b, sl]
                vxv = vxb[b, sl]
                vyv = vyb[b, sl]
                ucx = uxv + cv * vxv
                ucy = uyv + cv * vyv
                u2 = ucx * ucx + ucy * ucy + 1e-4
                ib = plsc.bitcast(u2, jnp.int32)
                y = plsc.bitcast(jnp.int32(0x5F3759DF) - (ib >> 1),
                                 jnp.float32)
                y = y * (1.5 - (0.5 * u2) * y * y)
                f = f + (vxv * vxv + vyv * vyv) * y
                s = s + (uxv * vxv + uyv * vyv) * y
                return f, s
            z = jnp.zeros((L,), jnp.float32)
            f, s = lax.fori_loop(0, NB, row, (z, z), unroll=False)
            cb[sl] = (accb[sl] - ALPHA * s) / (ALPHA * f + s1b[sl])
        return 0
    lax.fori_loop(0, 6, fp, 0, unroll=False)

    cp_dv.wait()
    for rl in range(RPW):
        il = rl // PR

        @plsc.parallel_loop(0, (NC * PC) // L, 1, unroll=2)
        def out_ch(ch):
            colv = ch * L + iota
            sl = pl.ds(ch * L, L)
            cg = plsc.load_gather(cb, [il * NC + colv // PC])
            dvv = u1f[pl.ds(rl * NC * PC + ch * L, L)]
            w1b[rl, sl] = w1b[rl, sl] + cg * dvv

    pltpu.async_copy(w1b, out_h.at[pl.ds(r0, RPW)], osem).wait()


def kernel(w1, noise_ch, nphi, vx, vy, dv1, s1, bnd_idx, bnd_idy):
    def pad48(col):
        return jnp.zeros((NBP,), jnp.int32).at[:NB].set(col.astype(jnp.int32))

    ix0, ix1 = pad48(bnd_idx[:, 0]), pad48(bnd_idx[:, 1])
    iy0, iy1 = pad48(bnd_idy[:, 0]), pad48(bnd_idy[:, 1])

    mesh = plsc.VectorSubcoreMesh(core_axis_name="c", subcore_axis_name="s",
                                  num_cores=2, num_subcores=16)
    run = pl.kernel(
        _mgmc_body,
        out_type=jax.ShapeDtypeStruct((NR * PR, NC * PC), jnp.float32),
        mesh=mesh,
        compiler_params=pltpu.CompilerParams(use_tc_tiling_on_sc=False,
                                             needs_layout_passes=False),
        scratch_types=[
            pltpu.VMEM((RPW, NC * PC), jnp.float32),
            pltpu.VMEM((Q * CPW,), jnp.float32),
            pltpu.VMEM((Q, CPW // 2), jnp.float32),
            pltpu.VMEM((Q, CPW // 2), jnp.float32),
            pltpu.VMEM((NB, CPW), jnp.float32),
            pltpu.VMEM((NB, CPW), jnp.float32),
            pltpu.VMEM((NB, CPW), jnp.float32),
            pltpu.VMEM((NB, CPW), jnp.float32),
            pltpu.VMEM((CPW,), jnp.float32),
            pltpu.VMEM((CPW,), jnp.float32),
            pltpu.VMEM((CPW,), jnp.float32),
            pltpu.VMEM((NBP,), jnp.int32),
            pltpu.VMEM((NBP,), jnp.int32),
            pltpu.VMEM((NBP,), jnp.int32),
            pltpu.VMEM((NBP,), jnp.int32),
            pltpu.SemaphoreType.DMA((7,)),
            pltpu.SemaphoreType.DMA,
        ],
    )
    return run(w1, noise_ch, nphi, vx, vy, dv1.reshape(-1), s1,
               ix0, ix1, iy0, iy1)


# speedup vs baseline: 1.9070x; 1.0054x over previous
import functools

import jax
import jax.numpy as jnp
from jax import lax
from jax.experimental import pallas as pl
from jax.experimental.pallas import tpu as pltpu
from jax.experimental.pallas import tpu_sc as plsc

PR, PC = 5, 5
NR, NC = 128, 128
NB = 40
ALPHA = 15.0
Q = PR * PC
L = 16
NWORK = 32
IPW = NR // NWORK
CPW = IPW * NC
RPW = IPW * PR
HC = CPW // 2
NBP = 48
W1C = NC * PC


def _splat(v):
    return jnp.full((L,), v, jnp.int32)


def _mgmc_body(w1_h, ns_h, np_h, vx_h, vy_h, dvf_h, s1_h, bnd_h, out_h,
               w1b, u1f, nsb, npb, vxb, vyb, uxb, uyb,
               s1b, accb, cb, bndb, cidx, sems, osem):
    wid = lax.axis_index("s") * 2 + lax.axis_index("c")
    iota = lax.iota(jnp.int32, L)

    pltpu.sync_copy(bnd_h, bndb)

    ii0 = wid * IPW
    r0 = ii0 * PR
    c0 = ii0 * NC

    cp_w1 = pltpu.async_copy(w1_h.at[pl.ds(r0, RPW)], w1b, sems.at[0])
    cp_ns = pltpu.async_copy(ns_h.at[:, pl.ds(c0, HC)], nsb, sems.at[1])
    cp_np = pltpu.async_copy(np_h.at[:, pl.ds(c0, HC)], npb, sems.at[2])
    cp_vx = pltpu.async_copy(vx_h.at[:, pl.ds(c0, CPW)], vxb, sems.at[3])
    cp_vy = pltpu.async_copy(vy_h.at[:, pl.ds(c0, CPW)], vyb, sems.at[4])
    cp_s1 = pltpu.async_copy(s1_h.at[pl.ds(c0, CPW)], s1b, sems.at[5])

    @plsc.parallel_loop(0, W1C // L, 1, unroll=2)
    def cidx_ch(ch):
        cidx[pl.ds(ch * L, L)] = (ch * L + iota) // PC

    cp_w1.wait()
    for h in range(2):
        if h == 0:
            cp_ns.wait()
            cp_np.wait()
        else:
            cp_ns2.wait()
            cp_np2.wait()

        @plsc.parallel_loop(0, HC // L, 1, unroll=2)
        def build_ch(ch):
            gch = h * (HC // L) + ch
            il = gch // (NC // L)
            jjv = (gch % (NC // L)) * L + iota
            base = il * PR
            a = jnp.zeros((L,), jnp.float32)
            for q in range(Q):
                rowv = _splat(base + q % PR)
                colv = jjv * PC + (q // PR)
                val = plsc.load_gather(w1b, [rowv, colv])
                u1f[pl.ds(q * CPW + gch * L, L)] = val
                a = a + (nsb[q, pl.ds(ch * L, L)] - val) * npb[q, pl.ds(ch * L, L)]
            accb[pl.ds(gch * L, L)] = a
            cb[pl.ds(gch * L, L)] = jnp.zeros((L,), jnp.float32)

        if h == 0:
            cp_ns2 = pltpu.async_copy(ns_h.at[:, pl.ds(c0 + HC, HC)],
                                      nsb, sems.at[1])
            cp_np2 = pltpu.async_copy(np_h.at[:, pl.ds(c0 + HC, HC)],
                                      npb, sems.at[2])

    def bld_row(b, _):
        ox0 = plsc.load_gather(bndb, [_splat(0), _splat(b)]) * CPW
        ox1 = plsc.load_gather(bndb, [_splat(1), _splat(b)]) * CPW
        oy0 = plsc.load_gather(bndb, [_splat(2), _splat(b)]) * CPW
        oy1 = plsc.load_gather(bndb, [_splat(3), _splat(b)]) * CPW

        @plsc.parallel_loop(0, CPW // L, 1, unroll=4)
        def bld_ch(ch):
            colv = ch * L + iota
            sl = pl.ds(ch * L, L)
            uxb[b, sl] = (plsc.load_gather(u1f, [ox0 + colv])
                          - plsc.load_gather(u1f, [ox1 + colv]))
            uyb[b, sl] = (plsc.load_gather(u1f, [oy0 + colv])
                          - plsc.load_gather(u1f, [oy1 + colv]))
        return 0
    lax.fori_loop(0, NB, bld_row, 0, unroll=False)

    cp_dv = pltpu.async_copy(dvf_h.at[pl.ds(r0 * W1C, RPW * W1C)],
                             u1f, sems.at[6])

    cp_vx.wait()
    cp_vy.wait()
    cp_s1.wait()

    def fp(it, _):
        @plsc.parallel_loop(0, CPW // L, 1, unroll=2)
        def ch_body(ch):
            sl = pl.ds(ch * L, L)
            cv = cb[sl]
            def row(b, carry):
                f, s = carry
                uxv = uxb[b, sl]
                uyv = uyb[b, sl]
                vxv = vxb[b, sl]
                vyv = vyb[b, sl]
                ucx = uxv + cv * vxv
                ucy = uyv + cv * vyv
                u2 = ucx * ucx + ucy * ucy + 1e-4
                ib = plsc.bitcast(u2, jnp.int32)
                y = plsc.bitcast(jnp.int32(0x5F3759DF) - (ib >> 1),
                                 jnp.float32)
                y = y * (1.5 - (0.5 * u2) * y * y)
                f = f + (vxv * vxv + vyv * vyv) * y
                s = s + (uxv * vxv + uyv * vyv) * y
                return f, s
            z = jnp.zeros((L,), jnp.float32)
            f, s = lax.fori_loop(0, NB, row, (z, z), unroll=False)
            cb[sl] = (accb[sl] - ALPHA * s) / (ALPHA * f + s1b[sl])
        return 0
    lax.fori_loop(0, 6, fp, 0, unroll=False)

    cp_dv.wait()
    for rl in range(RPW):
        il = rl // PR

        @plsc.parallel_loop(0, W1C // L, 1, unroll=4)
        def out_ch(ch):
            sl = pl.ds(ch * L, L)
            cg = plsc.load_gather(cb, [il * NC + cidx[sl]])
            dvv = u1f[pl.ds(rl * W1C + ch * L, L)]
            w1b[rl, sl] = w1b[rl, sl] + cg * dvv

    pltpu.async_copy(w1b, out_h.at[pl.ds(r0, RPW)], osem).wait()


def kernel(w1, noise_ch, nphi, vx, vy, dv1, s1, bnd_idx, bnd_idy):
    bnd = jnp.zeros((4, NBP), jnp.int32)
    bnd = bnd.at[0, :NB].set(bnd_idx[:, 0].astype(jnp.int32))
    bnd = bnd.at[1, :NB].set(bnd_idx[:, 1].astype(jnp.int32))
    bnd = bnd.at[2, :NB].set(bnd_idy[:, 0].astype(jnp.int32))
    bnd = bnd.at[3, :NB].set(bnd_idy[:, 1].astype(jnp.int32))

    mesh = plsc.VectorSubcoreMesh(core_axis_name="c", subcore_axis_name="s",
                                  num_cores=2, num_subcores=16)
    run = pl.kernel(
        _mgmc_body,
        out_type=jax.ShapeDtypeStruct((NR * PR, W1C), jnp.float32),
        mesh=mesh,
        compiler_params=pltpu.CompilerParams(use_tc_tiling_on_sc=False,
                                             needs_layout_passes=False),
        scratch_types=[
            pltpu.VMEM((RPW, W1C), jnp.float32),
            pltpu.VMEM((Q * CPW,), jnp.float32),
            pltpu.VMEM((Q, HC), jnp.float32),
            pltpu.VMEM((Q, HC), jnp.float32),
            pltpu.VMEM((NB, CPW), jnp.float32),
            pltpu.VMEM((NB, CPW), jnp.float32),
            pltpu.VMEM((NB, CPW), jnp.float32),
            pltpu.VMEM((NB, CPW), jnp.float32),
            pltpu.VMEM((CPW,), jnp.float32),
            pltpu.VMEM((CPW,), jnp.float32),
            pltpu.VMEM((CPW,), jnp.float32),
            pltpu.VMEM((4, NBP), jnp.int32),
            pltpu.VMEM((W1C,), jnp.int32),
            pltpu.SemaphoreType.DMA((7,)),
            pltpu.SemaphoreType.DMA,
        ],
    )
    return run(w1, noise_ch, nphi, vx, vy, dv1.reshape(-1), s1, bnd)


# speedup vs baseline: 2.6084x; 1.3678x over previous
import functools

import jax
import jax.numpy as jnp
from jax import lax
from jax.experimental import pallas as pl
from jax.experimental.pallas import tpu as pltpu
from jax.experimental.pallas import tpu_sc as plsc

PR, PC = 5, 5
NR, NC = 128, 128
NB = 40
ALPHA = 15.0
Q = PR * PC
W1C = NC * PC
NBP = 48

_KS = 64

L = 16
NWORK = 32
IPW = _KS // NWORK
CPW = IPW * NC
RPW = IPW * PR
HC = CPW // 2

TN = NR - _KS
TROWS = TN * PR


def _splat(v):
    return jnp.full((L,), v, jnp.int32)


def _sc_body(w1_h, ns_h, np_h, vx_h, vy_h, dvf_h, s1_h, bnd_h, out_h,
             w1b, u1f, nsb, npb, vxb, vyb, uxb, uyb,
             s1b, accb, cb, bndb, cidx, sems, osem):
    wid = lax.axis_index("s") * 2 + lax.axis_index("c")
    iota = lax.iota(jnp.int32, L)

    pltpu.sync_copy(bnd_h, bndb)

    ii0 = wid * IPW
    r0 = ii0 * PR
    c0 = ii0 * NC

    cp_w1 = pltpu.async_copy(w1_h.at[pl.ds(r0, RPW)], w1b, sems.at[0])
    cp_ns = pltpu.async_copy(ns_h.at[:, pl.ds(c0, HC)], nsb, sems.at[1])
    cp_np = pltpu.async_copy(np_h.at[:, pl.ds(c0, HC)], npb, sems.at[2])
    cp_vx = pltpu.async_copy(vx_h.at[:, pl.ds(c0, CPW)], vxb, sems.at[3])
    cp_vy = pltpu.async_copy(vy_h.at[:, pl.ds(c0, CPW)], vyb, sems.at[4])
    cp_s1 = pltpu.async_copy(s1_h.at[pl.ds(c0, CPW)], s1b, sems.at[5])

    @plsc.parallel_loop(0, W1C // L, 1, unroll=2)
    def cidx_ch(ch):
        cidx[pl.ds(ch * L, L)] = (ch * L + iota) // PC

    cp_w1.wait()
    for h in range(2):
        if h == 0:
            cp_ns.wait()
            cp_np.wait()
        else:
            cp_ns2.wait()
            cp_np2.wait()

        @plsc.parallel_loop(0, HC // L, 1, unroll=2)
        def build_ch(ch):
            gch = h * (HC // L) + ch
            il = gch // (NC // L)
            jjv = (gch % (NC // L)) * L + iota
            base = il * PR
            a = jnp.zeros((L,), jnp.float32)
            for q in range(Q):
                rowv = _splat(base + q % PR)
                colv = jjv * PC + (q // PR)
                val = plsc.load_gather(w1b, [rowv, colv])
                u1f[pl.ds(q * CPW + gch * L, L)] = val
                a = a + (nsb[q, pl.ds(ch * L, L)] - val) * npb[q, pl.ds(ch * L, L)]
            accb[pl.ds(gch * L, L)] = a
            cb[pl.ds(gch * L, L)] = jnp.zeros((L,), jnp.float32)

        if h == 0:
            cp_ns2 = pltpu.async_copy(ns_h.at[:, pl.ds(c0 + HC, HC)],
                                      nsb, sems.at[1])
            cp_np2 = pltpu.async_copy(np_h.at[:, pl.ds(c0 + HC, HC)],
                                      npb, sems.at[2])

    def bld_row(b, _):
        ox0 = plsc.load_gather(bndb, [_splat(0), _splat(b)]) * CPW
        ox1 = plsc.load_gather(bndb, [_splat(1), _splat(b)]) * CPW
        oy0 = plsc.load_gather(bndb, [_splat(2), _splat(b)]) * CPW
        oy1 = plsc.load_gather(bndb, [_splat(3), _splat(b)]) * CPW

        @plsc.parallel_loop(0, CPW // L, 1, unroll=4)
        def bld_ch(ch):
            colv = ch * L + iota
            sl = pl.ds(ch * L, L)
            uxb[b, sl] = (plsc.load_gather(u1f, [ox0 + colv])
                          - plsc.load_gather(u1f, [ox1 + colv]))
            uyb[b, sl] = (plsc.load_gather(u1f, [oy0 + colv])
                          - plsc.load_gather(u1f, [oy1 + colv]))
        return 0
    lax.fori_loop(0, NB, bld_row, 0, unroll=False)

    cp_dv = pltpu.async_copy(dvf_h.at[pl.ds(r0 * W1C, RPW * W1C)],
                             u1f, sems.at[6])

    cp_vx.wait()
    cp_vy.wait()
    cp_s1.wait()

    def fp(it, _):
        @plsc.parallel_loop(0, CPW // L, 1, unroll=2)
        def ch_body(ch):
            sl = pl.ds(ch * L, L)
            cv = cb[sl]
            def row(b, carry):
                f, s = carry
                uxv = uxb[b, sl]
                uyv = uyb[b, sl]
                vxv = vxb[b, sl]
                vyv = vyb[b, sl]
                ucx = uxv + cv * vxv
                ucy = uyv + cv * vyv
                u2 = ucx * ucx + ucy * ucy + 1e-4
                ib = plsc.bitcast(u2, jnp.int32)
                y = plsc.bitcast(jnp.int32(0x5F3759DF) - (ib >> 1),
                                 jnp.float32)
                y = y * (1.5 - (0.5 * u2) * y * y)
                f = f + (vxv * vxv + vyv * vyv) * y
                s = s + (uxv * vxv + uyv * vyv) * y
                return f, s
            z = jnp.zeros((L,), jnp.float32)
            f, s = lax.fori_loop(0, NB, row, (z, z), unroll=False)
            cb[sl] = (accb[sl] - ALPHA * s) / (ALPHA * f + s1b[sl])
        return 0
    lax.fori_loop(0, 6, fp, 0, unroll=False)

    cp_dv.wait()
    for rl in range(RPW):
        il = rl // PR

        @plsc.parallel_loop(0, W1C // L, 1, unroll=4)
        def out_ch(ch):
            sl = pl.ds(ch * L, L)
            cg = plsc.load_gather(cb, [il * NC + cidx[sl]])
            dvv = u1f[pl.ds(rl * W1C + ch * L, L)]
            w1b[rl, sl] = w1b[rl, sl] + cg * dvv

    pltpu.async_copy(w1b, out_h.at[pl.ds(r0, RPW)], osem).wait()


def _make_sc():
    mesh = plsc.VectorSubcoreMesh(core_axis_name="c", subcore_axis_name="s",
                                  num_cores=2, num_subcores=16)
    return pl.kernel(
        _sc_body,
        out_type=jax.ShapeDtypeStruct((_KS * PR, W1C), jnp.float32),
        mesh=mesh,
        compiler_params=pltpu.CompilerParams(use_tc_tiling_on_sc=False,
                                             needs_layout_passes=False),
        scratch_types=[
            pltpu.VMEM((RPW, W1C), jnp.float32),
            pltpu.VMEM((Q * CPW,), jnp.float32),
            pltpu.VMEM((Q, HC), jnp.float32),
            pltpu.VMEM((Q, HC), jnp.float32),
            pltpu.VMEM((NB, CPW), jnp.float32),
            pltpu.VMEM((NB, CPW), jnp.float32),
            pltpu.VMEM((NB, CPW), jnp.float32),
            pltpu.VMEM((NB, CPW), jnp.float32),
            pltpu.VMEM((CPW,), jnp.float32),
            pltpu.VMEM((CPW,), jnp.float32),
            pltpu.VMEM((CPW,), jnp.float32),
            pltpu.VMEM((4, NBP), jnp.int32),
            pltpu.VMEM((W1C,), jnp.int32),
            pltpu.SemaphoreType.DMA((7,)),
            pltpu.SemaphoreType.DMA,
        ],
    )


def _tc_body(bnd_s, w1r, nsr, npr, vxr, vyr, dvr, s1r, outr, uscr):
    cidx = lax.broadcasted_iota(jnp.int32, (W1C, NC), 0)
    jidx = lax.broadcasted_iota(jnp.int32, (W1C, NC), 1)
    ridx5 = lax.broadcasted_iota(jnp.int32, (TN, TROWS), 1)
    kidx5 = lax.broadcasted_iota(jnp.int32, (TN, TROWS), 0)

    w1v = w1r[...]
    for g in range(PC):
        csel = (cidx == jidx * PC + g).astype(jnp.float32)
        ag = jnp.dot(w1v, csel, preferred_element_type=jnp.float32)
        for r in range(PR):
            srow = (ridx5 == kidx5 * PR + r).astype(jnp.float32)
            uscr[g * PR + r] = jnp.dot(srow, ag,
                                       preferred_element_type=jnp.float32)

    acc = jnp.zeros((TN, NC), jnp.float32)
    for q in range(Q):
        acc = acc + (nsr[q] - uscr[q]) * npr[q]

    ux = [uscr[bnd_s[0, b]] - uscr[bnd_s[1, b]] for b in range(NB)]
    uy = [uscr[bnd_s[2, b]] - uscr[bnd_s[3, b]] for b in range(NB)]
    vxl = [vxr[b] for b in range(NB)]
    vyl = [vyr[b] for b in range(NB)]
    a1 = [vxl[b] * vxl[b] + vyl[b] * vyl[b] for b in range(NB)]
    c1 = [ux[b] * vxl[b] + uy[b] * vyl[b] for b in range(NB)]

    s1v = s1r[...]
    c = jnp.zeros((TN, NC), jnp.float32)
    for _ in range(6):
        firs = jnp.zeros((TN, NC), jnp.float32)
        sec = jnp.zeros((TN, NC), jnp.float32)
        for b in range(NB):
            ucx = ux[b] + c * vxl[b]
            ucy = uy[b] + c * vyl[b]
            rb = lax.rsqrt(ucx * ucx + ucy * ucy + 1e-4)
            firs = firs + a1[b] * rb
            sec = sec + c1[b] * rb
        c = (acc - ALPHA * sec) / (ALPHA * firs + s1v)

    brow = (lax.broadcasted_iota(jnp.int32, (TROWS, TN), 0) // PR
            == lax.broadcasted_iota(jnp.int32, (TROWS, TN), 1)
            ).astype(jnp.float32)
    bcol = (lax.broadcasted_iota(jnp.int32, (NC, W1C), 1) // PC
            == lax.broadcasted_iota(jnp.int32, (NC, W1C), 0)
            ).astype(jnp.float32)
    dc = jnp.dot(jnp.dot(brow, c, preferred_element_type=jnp.float32),
                 bcol, preferred_element_type=jnp.float32)
    outr[...] = w1v + dc * dvr[...]


def _make_tc():
    grid_spec = pltpu.PrefetchScalarGridSpec(
        num_scalar_prefetch=1,
        grid=(1,),
        in_specs=[
            pl.BlockSpec((TROWS, W1C), lambda i, b: (0, 0)),
            pl.BlockSpec((Q, TN, NC), lambda i, b: (0, 0, 0)),
            pl.BlockSpec((Q, TN, NC), lambda i, b: (0, 0, 0)),
            pl.BlockSpec((NB, TN, NC), lambda i, b: (0, 0, 0)),
            pl.BlockSpec((NB, TN, NC), lambda i, b: (0, 0, 0)),
            pl.BlockSpec((TROWS, W1C), lambda i, b: (0, 0)),
            pl.BlockSpec((TN, NC), lambda i, b: (0, 0)),
        ],
        out_specs=pl.BlockSpec((TROWS, W1C), lambda i, b: (0, 0)),
        scratch_shapes=[pltpu.VMEM((Q, TN, NC), jnp.float32)],
    )
    return pl.pallas_call(
        _tc_body,
        grid_spec=grid_spec,
        out_shape=jax.ShapeDtypeStruct((TROWS, W1C), jnp.float32),
        compiler_params=pltpu.CompilerParams(
            vmem_limit_bytes=128 * 1024 * 1024),
    )


def kernel(w1, noise_ch, nphi, vx, vy, dv1, s1, bnd_idx, bnd_idy):
    bnd = jnp.zeros((4, NBP), jnp.int32)
    bnd = bnd.at[0, :NB].set(bnd_idx[:, 0].astype(jnp.int32))
    bnd = bnd.at[1, :NB].set(bnd_idx[:, 1].astype(jnp.int32))
    bnd = bnd.at[2, :NB].set(bnd_idy[:, 0].astype(jnp.int32))
    bnd = bnd.at[3, :NB].set(bnd_idy[:, 1].astype(jnp.int32))

    sc_out = _make_sc()(w1, noise_ch, nphi, vx, vy, dv1.reshape(-1), s1, bnd)

    r0 = _KS * PR
    c0 = _KS * NC
    tc_out = _make_tc()(
        bnd,
        w1[r0:],
        noise_ch[:, c0:].reshape(Q, TN, NC),
        nphi[:, c0:].reshape(Q, TN, NC),
        vx[:, c0:].reshape(NB, TN, NC),
        vy[:, c0:].reshape(NB, TN, NC),
        dv1[r0:],
        s1[c0:].reshape(TN, NC),
    )
    return jnp.concatenate([sc_out, tc_out], axis=0)


# speedup vs baseline: 3.4447x; 1.3206x over previous
import functools

import jax
import jax.numpy as jnp
from jax import lax
from jax.experimental import pallas as pl
from jax.experimental.pallas import tpu as pltpu
from jax.experimental.pallas import tpu_sc as plsc

PR, PC = 5, 5
NR, NC = 128, 128
NB = 40
ALPHA = 15.0
Q = PR * PC
W1C = NC * PC
NBP = 48

_KS = 32

L = 16
NWORK = 32
IPW = _KS // NWORK
CPW = IPW * NC
RPW = IPW * PR
HC = CPW // 2

TN = NR - _KS
TROWS = TN * PR


def _splat(v):
    return jnp.full((L,), v, jnp.int32)


def _sc_body(w1_h, ns_h, np_h, vx_h, vy_h, dvf_h, s1_h, bnd_h, out_h,
             w1b, u1f, nsb, npb, vxb, vyb, uxb, uyb,
             s1b, accb, cb, bndb, cidx, sems, osem):
    wid = lax.axis_index("s") * 2 + lax.axis_index("c")
    iota = lax.iota(jnp.int32, L)

    pltpu.sync_copy(bnd_h, bndb)

    ii0 = wid * IPW
    r0 = ii0 * PR
    c0 = ii0 * NC

    cp_w1 = pltpu.async_copy(w1_h.at[pl.ds(r0, RPW)], w1b, sems.at[0])
    cp_ns = pltpu.async_copy(ns_h.at[:, pl.ds(c0, HC)], nsb, sems.at[1])
    cp_np = pltpu.async_copy(np_h.at[:, pl.ds(c0, HC)], npb, sems.at[2])
    cp_vx = pltpu.async_copy(vx_h.at[:, pl.ds(c0, CPW)], vxb, sems.at[3])
    cp_vy = pltpu.async_copy(vy_h.at[:, pl.ds(c0, CPW)], vyb, sems.at[4])
    cp_s1 = pltpu.async_copy(s1_h.at[pl.ds(c0, CPW)], s1b, sems.at[5])

    @plsc.parallel_loop(0, W1C // L, 1, unroll=2)
    def cidx_ch(ch):
        cidx[pl.ds(ch * L, L)] = (ch * L + iota) // PC

    cp_w1.wait()
    for h in range(2):
        if h == 0:
            cp_ns.wait()
            cp_np.wait()
        else:
            cp_ns2.wait()
            cp_np2.wait()

        @plsc.parallel_loop(0, HC // L, 1, unroll=2)
        def build_ch(ch):
            gch = h * (HC // L) + ch
            il = gch // (NC // L)
            jjv = (gch % (NC // L)) * L + iota
            base = il * PR
            a = jnp.zeros((L,), jnp.float32)
            for q in range(Q):
                rowv = _splat(base + q % PR)
                colv = jjv * PC + (q // PR)
                val = plsc.load_gather(w1b, [rowv, colv])
                u1f[pl.ds(q * CPW + gch * L, L)] = val
                a = a + (nsb[q, pl.ds(ch * L, L)] - val) * npb[q, pl.ds(ch * L, L)]
            accb[pl.ds(gch * L, L)] = a
            cb[pl.ds(gch * L, L)] = jnp.zeros((L,), jnp.float32)

        if h == 0:
            cp_ns2 = pltpu.async_copy(ns_h.at[:, pl.ds(c0 + HC, HC)],
                                      nsb, sems.at[1])
            cp_np2 = pltpu.async_copy(np_h.at[:, pl.ds(c0 + HC, HC)],
                                      npb, sems.at[2])

    def bld_row(b, _):
        ox0 = plsc.load_gather(bndb, [_splat(0), _splat(b)]) * CPW
        ox1 = plsc.load_gather(bndb, [_splat(1), _splat(b)]) * CPW
        oy0 = plsc.load_gather(bndb, [_splat(2), _splat(b)]) * CPW
        oy1 = plsc.load_gather(bndb, [_splat(3), _splat(b)]) * CPW

        @plsc.parallel_loop(0, CPW // L, 1, unroll=4)
        def bld_ch(ch):
            colv = ch * L + iota
            sl = pl.ds(ch * L, L)
            uxb[b, sl] = (plsc.load_gather(u1f, [ox0 + colv])
                          - plsc.load_gather(u1f, [ox1 + colv]))
            uyb[b, sl] = (plsc.load_gather(u1f, [oy0 + colv])
                          - plsc.load_gather(u1f, [oy1 + colv]))
        return 0
    lax.fori_loop(0, NB, bld_row, 0, unroll=False)

    cp_dv = pltpu.async_copy(dvf_h.at[pl.ds(r0 * W1C, RPW * W1C)],
                             u1f, sems.at[6])

    cp_vx.wait()
    cp_vy.wait()
    cp_s1.wait()

    def fp(it, _):
        @plsc.parallel_loop(0, CPW // L, 1, unroll=2)
        def ch_body(ch):
            sl = pl.ds(ch * L, L)
            cv = cb[sl]
            def row(b, carry):
                f, s = carry
                uxv = uxb[b, sl]
                uyv = uyb[b, sl]
                vxv = vxb[b, sl]
                vyv = vyb[b, sl]
                ucx = uxv + cv * vxv
                ucy = uyv + cv * vyv
                u2 = ucx * ucx + ucy * ucy + 1e-4
                ib = plsc.bitcast(u2, jnp.int32)
                y = plsc.bitcast(jnp.int32(0x5F3759DF) - (ib >> 1),
                                 jnp.float32)
                y = y * (1.5 - (0.5 * u2) * y * y)
                f = f + (vxv * vxv + vyv * vyv) * y
                s = s + (uxv * vxv + uyv * vyv) * y
                return f, s
            z = jnp.zeros((L,), jnp.float32)
            f, s = lax.fori_loop(0, NB, row, (z, z), unroll=False)
            cb[sl] = (accb[sl] - ALPHA * s) / (ALPHA * f + s1b[sl])
        return 0
    lax.fori_loop(0, 6, fp, 0, unroll=False)

    cp_dv.wait()
    for rl in range(RPW):
        il = rl // PR

        @plsc.parallel_loop(0, W1C // L, 1, unroll=4)
        def out_ch(ch):
            sl = pl.ds(ch * L, L)
            cg = plsc.load_gather(cb, [il * NC + cidx[sl]])
            dvv = u1f[pl.ds(rl * W1C + ch * L, L)]
            w1b[rl, sl] = w1b[rl, sl] + cg * dvv

    pltpu.async_copy(w1b, out_h.at[pl.ds(r0, RPW)], osem).wait()


def _make_sc():
    mesh = plsc.VectorSubcoreMesh(core_axis_name="c", subcore_axis_name="s",
                                  num_cores=2, num_subcores=16)
    return pl.kernel(
        _sc_body,
        out_type=jax.ShapeDtypeStruct((_KS * PR, W1C), jnp.float32),
        mesh=mesh,
        compiler_params=pltpu.CompilerParams(use_tc_tiling_on_sc=False,
                                             needs_layout_passes=False),
        scratch_types=[
            pltpu.VMEM((RPW, W1C), jnp.float32),
            pltpu.VMEM((Q * CPW,), jnp.float32),
            pltpu.VMEM((Q, HC), jnp.float32),
            pltpu.VMEM((Q, HC), jnp.float32),
            pltpu.VMEM((NB, CPW), jnp.float32),
            pltpu.VMEM((NB, CPW), jnp.float32),
            pltpu.VMEM((NB, CPW), jnp.float32),
            pltpu.VMEM((NB, CPW), jnp.float32),
            pltpu.VMEM((CPW,), jnp.float32),
            pltpu.VMEM((CPW,), jnp.float32),
            pltpu.VMEM((CPW,), jnp.float32),
            pltpu.VMEM((4, NBP), jnp.int32),
            pltpu.VMEM((W1C,), jnp.int32),
            pltpu.SemaphoreType.DMA((7,)),
            pltpu.SemaphoreType.DMA,
        ],
    )


def _tc_body(bnd_s, w1r, nsr, npr, vxr, vyr, dvr, s1r, outr, uscr):
    cidx = lax.broadcasted_iota(jnp.int32, (W1C, NC), 0)
    jidx = lax.broadcasted_iota(jnp.int32, (W1C, NC), 1)
    ridx5 = lax.broadcasted_iota(jnp.int32, (TN, TROWS), 1)
    kidx5 = lax.broadcasted_iota(jnp.int32, (TN, TROWS), 0)

    w1v = w1r[...]
    for g in range(PC):
        csel = (cidx == jidx * PC + g).astype(jnp.float32)
        ag = jnp.dot(w1v, csel, preferred_element_type=jnp.float32)
        for r in range(PR):
            srow = (ridx5 == kidx5 * PR + r).astype(jnp.float32)
            uscr[g * PR + r] = jnp.dot(srow, ag,
                                       preferred_element_type=jnp.float32)

    acc = jnp.zeros((TN, NC), jnp.float32)
    for q in range(Q):
        acc = acc + (nsr[q] - uscr[q]) * npr[q]

    ux = [uscr[bnd_s[0, b]] - uscr[bnd_s[1, b]] for b in range(NB)]
    uy = [uscr[bnd_s[2, b]] - uscr[bnd_s[3, b]] for b in range(NB)]
    vxl = [vxr[b] for b in range(NB)]
    vyl = [vyr[b] for b in range(NB)]
    a1 = [vxl[b] * vxl[b] + vyl[b] * vyl[b] for b in range(NB)]
    c1 = [ux[b] * vxl[b] + uy[b] * vyl[b] for b in range(NB)]

    s1v = s1r[...]
    c = jnp.zeros((TN, NC), jnp.float32)
    for _ in range(6):
        firs = jnp.zeros((TN, NC), jnp.float32)
        sec = jnp.zeros((TN, NC), jnp.float32)
        for b in range(NB):
            ucx = ux[b] + c * vxl[b]
            ucy = uy[b] + c * vyl[b]
            rb = lax.rsqrt(ucx * ucx + ucy * ucy + 1e-4)
            firs = firs + a1[b] * rb
            sec = sec + c1[b] * rb
        c = (acc - ALPHA * sec) / (ALPHA * firs + s1v)

    brow = (lax.broadcasted_iota(jnp.int32, (TROWS, TN), 0) // PR
            == lax.broadcasted_iota(jnp.int32, (TROWS, TN), 1)
            ).astype(jnp.float32)
    bcol = (lax.broadcasted_iota(jnp.int32, (NC, W1C), 1) // PC
            == lax.broadcasted_iota(jnp.int32, (NC, W1C), 0)
            ).astype(jnp.float32)
    dc = jnp.dot(jnp.dot(brow, c, preferred_element_type=jnp.float32),
                 bcol, preferred_element_type=jnp.float32)
    outr[...] = w1v + dc * dvr[...]


def _make_tc():
    grid_spec = pltpu.PrefetchScalarGridSpec(
        num_scalar_prefetch=1,
        grid=(1,),
        in_specs=[
            pl.BlockSpec((TROWS, W1C), lambda i, b: (0, 0)),
            pl.BlockSpec((Q, TN, NC), lambda i, b: (0, 0, 0)),
            pl.BlockSpec((Q, TN, NC), lambda i, b: (0, 0, 0)),
            pl.BlockSpec((NB, TN, NC), lambda i, b: (0, 0, 0)),
            pl.BlockSpec((NB, TN, NC), lambda i, b: (0, 0, 0)),
            pl.BlockSpec((TROWS, W1C), lambda i, b: (0, 0)),
            pl.BlockSpec((TN, NC), lambda i, b: (0, 0)),
        ],
        out_specs=pl.BlockSpec((TROWS, W1C), lambda i, b: (0, 0)),
        scratch_shapes=[pltpu.VMEM((Q, TN, NC), jnp.float32)],
    )
    return pl.pallas_call(
        _tc_body,
        grid_spec=grid_spec,
        out_shape=jax.ShapeDtypeStruct((TROWS, W1C), jnp.float32),
        compiler_params=pltpu.CompilerParams(
            vmem_limit_bytes=128 * 1024 * 1024),
    )


def kernel(w1, noise_ch, nphi, vx, vy, dv1, s1, bnd_idx, bnd_idy):
    bnd = jnp.zeros((4, NBP), jnp.int32)
    bnd = bnd.at[0, :NB].set(bnd_idx[:, 0].astype(jnp.int32))
    bnd = bnd.at[1, :NB].set(bnd_idx[:, 1].astype(jnp.int32))
    bnd = bnd.at[2, :NB].set(bnd_idy[:, 0].astype(jnp.int32))
    bnd = bnd.at[3, :NB].set(bnd_idy[:, 1].astype(jnp.int32))

    r0 = _KS * PR
    c0 = _KS * NC
    sc_out = _make_sc()(w1[:r0], noise_ch[:, :c0], nphi[:, :c0],
                        vx[:, :c0], vy[:, :c0], dv1[:r0].reshape(-1),
                        s1[:c0], bnd)
    tc_out = _make_tc()(
        bnd,
        w1[r0:],
        noise_ch[:, c0:].reshape(Q, TN, NC),
        nphi[:, c0:].reshape(Q, TN, NC),
        vx[:, c0:].reshape(NB, TN, NC),
        vy[:, c0:].reshape(NB, TN, NC),
        dv1[r0:],
        s1[c0:].reshape(TN, NC),
    )
    return jnp.concatenate([sc_out, tc_out], axis=0)


# speedup vs baseline: 3.7052x; 1.0756x over previous
import functools

import jax
import jax.numpy as jnp
from jax import lax
from jax.experimental import pallas as pl
from jax.experimental.pallas import tpu as pltpu
from jax.experimental.pallas import tpu_sc as plsc

PR, PC = 5, 5
NR, NC = 128, 128
NB = 40
ALPHA = 15.0
Q = PR * PC
W1C = NC * PC
NBP = 48

_KS = 32

L = 16
NWORK = 32
IPW = _KS // NWORK
CPW = IPW * NC
RPW = IPW * PR
HC = CPW // 2

TN = NR - _KS
TROWS = TN * PR


def _splat(v):
    return jnp.full((L,), v, jnp.int32)


def _sc_body(w1_h, ns_h, np_h, vx_h, vy_h, dvf_h, s1_h, bnd_h, out_h,
             w1b, u1f, nsb, npb, vxb, vyb, uxb, uyb,
             s1b, accb, cb, bndb, cidx, sems, osem):
    wid = lax.axis_index("s") * 2 + lax.axis_index("c")
    iota = lax.iota(jnp.int32, L)

    pltpu.sync_copy(bnd_h, bndb)

    ii0 = wid * IPW
    r0 = ii0 * PR
    c0 = ii0 * NC

    cp_w1 = pltpu.async_copy(w1_h.at[pl.ds(r0, RPW)], w1b, sems.at[0])
    cp_ns = pltpu.async_copy(ns_h.at[:, pl.ds(c0, HC)], nsb, sems.at[1])
    cp_np = pltpu.async_copy(np_h.at[:, pl.ds(c0, HC)], npb, sems.at[2])
    cp_vx = pltpu.async_copy(vx_h.at[:, pl.ds(c0, CPW)], vxb, sems.at[3])
    cp_vy = pltpu.async_copy(vy_h.at[:, pl.ds(c0, CPW)], vyb, sems.at[4])
    cp_s1 = pltpu.async_copy(s1_h.at[pl.ds(c0, CPW)], s1b, sems.at[5])

    @plsc.parallel_loop(0, W1C // L, 1, unroll=2)
    def cidx_ch(ch):
        cidx[pl.ds(ch * L, L)] = (ch * L + iota) // PC

    cp_w1.wait()
    for h in range(2):
        if h == 0:
            cp_ns.wait()
            cp_np.wait()
        else:
            cp_ns2.wait()
            cp_np2.wait()

        @plsc.parallel_loop(0, HC // L, 1, unroll=2)
        def build_ch(ch):
            gch = h * (HC // L) + ch
            il = gch // (NC // L)
            jjv = (gch % (NC // L)) * L + iota
            base = il * PR
            a = jnp.zeros((L,), jnp.float32)
            for q in range(Q):
                rowv = _splat(base + q % PR)
                colv = jjv * PC + (q // PR)
                val = plsc.load_gather(w1b, [rowv, colv])
                u1f[pl.ds(q * CPW + gch * L, L)] = val
                a = a + (nsb[q, pl.ds(ch * L, L)] - val) * npb[q, pl.ds(ch * L, L)]
            accb[pl.ds(gch * L, L)] = a
            cb[pl.ds(gch * L, L)] = jnp.zeros((L,), jnp.float32)

        if h == 0:
            cp_ns2 = pltpu.async_copy(ns_h.at[:, pl.ds(c0 + HC, HC)],
                                      nsb, sems.at[1])
            cp_np2 = pltpu.async_copy(np_h.at[:, pl.ds(c0 + HC, HC)],
                                      npb, sems.at[2])

    def bld_row(b, _):
        ox0 = plsc.load_gather(bndb, [_splat(0), _splat(b)]) * CPW
        ox1 = plsc.load_gather(bndb, [_splat(1), _splat(b)]) * CPW
        oy0 = plsc.load_gather(bndb, [_splat(2), _splat(b)]) * CPW
        oy1 = plsc.load_gather(bndb, [_splat(3), _splat(b)]) * CPW

        @plsc.parallel_loop(0, CPW // L, 1, unroll=4)
        def bld_ch(ch):
            colv = ch * L + iota
            sl = pl.ds(ch * L, L)
            uxb[b, sl] = (plsc.load_gather(u1f, [ox0 + colv])
                          - plsc.load_gather(u1f, [ox1 + colv]))
            uyb[b, sl] = (plsc.load_gather(u1f, [oy0 + colv])
                          - plsc.load_gather(u1f, [oy1 + colv]))
        return 0
    lax.fori_loop(0, NB, bld_row, 0, unroll=False)

    cp_dv = pltpu.async_copy(dvf_h.at[pl.ds(r0 * W1C, RPW * W1C)],
                             u1f, sems.at[6])

    cp_vx.wait()
    cp_vy.wait()
    cp_s1.wait()

    def fp(it, _):
        @plsc.parallel_loop(0, CPW // L, 1, unroll=2)
        def ch_body(ch):
            sl = pl.ds(ch * L, L)
            cv = cb[sl]
            def row(b, carry):
                f, s = carry
                uxv = uxb[b, sl]
                uyv = uyb[b, sl]
                vxv = vxb[b, sl]
                vyv = vyb[b, sl]
                ucx = uxv + cv * vxv
                ucy = uyv + cv * vyv
                u2 = ucx * ucx + ucy * ucy + 1e-4
                ib = plsc.bitcast(u2, jnp.int32)
                y = plsc.bitcast(jnp.int32(0x5F3759DF) - (ib >> 1),
                                 jnp.float32)
                y = y * (1.5 - (0.5 * u2) * y * y)
                f = f + (vxv * vxv + vyv * vyv) * y
                s = s + (uxv * vxv + uyv * vyv) * y
                return f, s
            z = jnp.zeros((L,), jnp.float32)
            f, s = lax.fori_loop(0, NB, row, (z, z), unroll=False)
            cb[sl] = (accb[sl] - ALPHA * s) / (ALPHA * f + s1b[sl])
        return 0
    lax.fori_loop(0, 6, fp, 0, unroll=False)

    cp_dv.wait()
    for rl in range(RPW):
        il = rl // PR

        @plsc.parallel_loop(0, W1C // L, 1, unroll=4)
        def out_ch(ch):
            sl = pl.ds(ch * L, L)
            cg = plsc.load_gather(cb, [il * NC + cidx[sl]])
            dvv = u1f[pl.ds(rl * W1C + ch * L, L)]
            w1b[rl, sl] = w1b[rl, sl] + cg * dvv

    pltpu.async_copy(w1b, out_h.at[pl.ds(r0, RPW)], osem).wait()


def _make_sc():
    mesh = plsc.VectorSubcoreMesh(core_axis_name="c", subcore_axis_name="s",
                                  num_cores=2, num_subcores=16)
    return pl.kernel(
        _sc_body,
        out_type=jax.ShapeDtypeStruct((_KS * PR, W1C), jnp.float32),
        mesh=mesh,
        compiler_params=pltpu.CompilerParams(use_tc_tiling_on_sc=False,
                                             needs_layout_passes=False),
        scratch_types=[
            pltpu.VMEM((RPW, W1C), jnp.float32),
            pltpu.VMEM((Q * CPW,), jnp.float32),
            pltpu.VMEM((Q, HC), jnp.float32),
            pltpu.VMEM((Q, HC), jnp.float32),
            pltpu.VMEM((NB, CPW), jnp.float32),
            pltpu.VMEM((NB, CPW), jnp.float32),
            pltpu.VMEM((NB, CPW), jnp.float32),
            pltpu.VMEM((NB, CPW), jnp.float32),
            pltpu.VMEM((CPW,), jnp.float32),
            pltpu.VMEM((CPW,), jnp.float32),
            pltpu.VMEM((CPW,), jnp.float32),
            pltpu.VMEM((4, NBP), jnp.int32),
            pltpu.VMEM((W1C,), jnp.int32),
            pltpu.SemaphoreType.DMA((7,)),
            pltpu.SemaphoreType.DMA,
        ],
    )


BI = 32
NSTEP = TN // BI
OFF = _KS // BI
BC = BI * NC
BR = BI * PR


def _tc_body(w1r, nsr, npr, vxr, vyr, dvr, s1r, sxr, syr, outr, uscr, cscr):
    cidx = lax.broadcasted_iota(jnp.int32, (W1C, NC), 0)
    jidx = lax.broadcasted_iota(jnp.int32, (W1C, NC), 1)
    w1v = w1r[...]
    for g in range(PC):
        csel = (cidx == jidx * PC + g).astype(jnp.float32)
        ag = jnp.dot(w1v, csel, preferred_element_type=jnp.float32)
        for il in range(BI):
            uscr[pl.ds(g * PR, PR), pl.ds(il * NC, NC)] = (
                ag[il * PR:(il + 1) * PR, :])

    uall = uscr[...]
    acc = jnp.sum((nsr[...] - uall) * npr[...], axis=0, keepdims=True)

    ux = jnp.dot(sxr[...], uall, preferred_element_type=jnp.float32)
    uy = jnp.dot(syr[...], uall, preferred_element_type=jnp.float32)

    vxv = vxr[...]
    vyv = vyr[...]
    a1 = vxv * vxv + vyv * vyv
    c1 = ux * vxv + uy * vyv
    s1v = s1r[...].reshape(1, BC)

    c = jnp.zeros((1, BC), jnp.float32)
    for _ in range(6):
        ucx = ux + c * vxv
        ucy = uy + c * vyv
        rb = lax.rsqrt(ucx * ucx + ucy * ucy + 1e-4)
        firs = jnp.sum(a1 * rb, axis=0, keepdims=True)
        sec = jnp.sum(c1 * rb, axis=0, keepdims=True)
        c = (acc - ALPHA * sec) / (ALPHA * firs + s1v)

    for il in range(BI):
        cscr[il, :] = c[0, il * NC:(il + 1) * NC]
    brow = (lax.broadcasted_iota(jnp.int32, (BR, BI), 0) // PR
            == lax.broadcasted_iota(jnp.int32, (BR, BI), 1)
            ).astype(jnp.float32)
    bcol = (lax.broadcasted_iota(jnp.int32, (NC, W1C), 1) // PC
            == lax.broadcasted_iota(jnp.int32, (NC, W1C), 0)
            ).astype(jnp.float32)
    dc = jnp.dot(jnp.dot(brow, cscr[...], preferred_element_type=jnp.float32),
                 bcol, preferred_element_type=jnp.float32)
    outr[...] = w1v + dc * dvr[...]


def _make_tc():
    return pl.pallas_call(
        _tc_body,
        grid=(NSTEP,),
        in_specs=[
            pl.BlockSpec((BR, W1C), lambda i: (OFF + i, 0)),
            pl.BlockSpec((Q, BC), lambda i: (0, OFF + i)),
            pl.BlockSpec((Q, BC), lambda i: (0, OFF + i)),
            pl.BlockSpec((NB, BC), lambda i: (0, OFF + i)),
            pl.BlockSpec((NB, BC), lambda i: (0, OFF + i)),
            pl.BlockSpec((BR, W1C), lambda i: (OFF + i, 0)),
            pl.BlockSpec((BC,), lambda i: (OFF + i,)),
            pl.BlockSpec((NB, Q), lambda i: (0, 0)),
            pl.BlockSpec((NB, Q), lambda i: (0, 0)),
        ],
        out_specs=pl.BlockSpec((BR, W1C), lambda i: (i, 0)),
        out_shape=jax.ShapeDtypeStruct((TN * PR, W1C), jnp.float32),
        scratch_shapes=[pltpu.VMEM((Q, BC), jnp.float32),
                        pltpu.VMEM((BI, NC), jnp.float32)],
        compiler_params=pltpu.CompilerParams(
            dimension_semantics=("arbitrary",),
            vmem_limit_bytes=110 * 1024 * 1024),
    )


def kernel(w1, noise_ch, nphi, vx, vy, dv1, s1, bnd_idx, bnd_idy):
    bnd = jnp.zeros((4, NBP), jnp.int32)
    bnd = bnd.at[0, :NB].set(bnd_idx[:, 0].astype(jnp.int32))
    bnd = bnd.at[1, :NB].set(bnd_idx[:, 1].astype(jnp.int32))
    bnd = bnd.at[2, :NB].set(bnd_idy[:, 0].astype(jnp.int32))
    bnd = bnd.at[3, :NB].set(bnd_idy[:, 1].astype(jnp.int32))

    r0 = _KS * PR
    c0 = _KS * NC
    sc_out = _make_sc()(w1[:r0], noise_ch[:, :c0], nphi[:, :c0],
                        vx[:, :c0], vy[:, :c0], dv1[:r0].reshape(-1),
                        s1[:c0], bnd)
    qio = jnp.arange(Q, dtype=jnp.int32)[None, :]
    sx = ((qio == bnd[0, :NB, None]).astype(jnp.float32)
          - (qio == bnd[1, :NB, None]).astype(jnp.float32))
    sy = ((qio == bnd[2, :NB, None]).astype(jnp.float32)
          - (qio == bnd[3, :NB, None]).astype(jnp.float32))
    tc_out = _make_tc()(w1, noise_ch, nphi, vx, vy, dv1, s1, sx, sy)
    return jnp.concatenate([sc_out, tc_out], axis=0)


# speedup vs baseline: 4.1288x; 1.1143x over previous
import functools

import jax
import jax.numpy as jnp
from jax import lax
from jax.experimental import pallas as pl
from jax.experimental.pallas import tpu as pltpu
from jax.experimental.pallas import tpu_sc as plsc

PR, PC = 5, 5
NR, NC = 128, 128
NB = 40
ALPHA = 15.0
Q = PR * PC
W1C = NC * PC
NBP = 48

_KS = 16

L = 16
NWORK = 32
CPW = NC // 2
RPW = PR
WCW = CPW * PC
HC = CPW // 2

TN = NR - _KS
TROWS = TN * PR


def _splat(v):
    return jnp.full((L,), v, jnp.int32)


def _sc_body(w1_h, ns_h, np_h, vx_h, vy_h, dv_h, s1_h, bnd_h, out_h,
             w1b, u1f, dvb, nsb, npb, vxb, vyb, uxb, uyb,
             s1b, accb, cb, bndb, cidx, sems, osem):
    wid = lax.axis_index("s") * 2 + lax.axis_index("c")
    iota = lax.iota(jnp.int32, L)

    pltpu.sync_copy(bnd_h, bndb)

    ii = wid // 2
    jh = wid % 2
    r0 = ii * PR
    c0 = ii * NC + jh * CPW
    w0 = jh * WCW

    cp_w1 = pltpu.async_copy(w1_h.at[pl.ds(r0, RPW), pl.ds(w0, WCW)],
                             w1b, sems.at[0])
    cp_ns = pltpu.async_copy(ns_h.at[:, pl.ds(c0, HC)], nsb, sems.at[1])
    cp_np = pltpu.async_copy(np_h.at[:, pl.ds(c0, HC)], npb, sems.at[2])
    cp_vx = pltpu.async_copy(vx_h.at[:, pl.ds(c0, CPW)], vxb, sems.at[3])
    cp_vy = pltpu.async_copy(vy_h.at[:, pl.ds(c0, CPW)], vyb, sems.at[4])
    cp_s1 = pltpu.async_copy(s1_h.at[pl.ds(c0, CPW)], s1b, sems.at[5])
    cp_dv = pltpu.async_copy(dv_h.at[pl.ds(r0, RPW), pl.ds(w0, WCW)],
                             dvb, sems.at[6])

    @plsc.parallel_loop(0, WCW // L, 1, unroll=2)
    def cidx_ch(ch):
        cidx[pl.ds(ch * L, L)] = (ch * L + iota) // PC

    cp_w1.wait()
    for h in range(2):
        if h == 0:
            cp_ns.wait()
            cp_np.wait()
        else:
            cp_ns2.wait()
            cp_np2.wait()

        @plsc.parallel_loop(0, HC // L, 1, unroll=2)
        def build_ch(ch):
            gch = h * (HC // L) + ch
            jjv = gch * L + iota
            a = jnp.zeros((L,), jnp.float32)
            for q in range(Q):
                rowv = _splat(q % PR)
                colv = jjv * PC + (q // PR)
                val = plsc.load_gather(w1b, [rowv, colv])
                u1f[pl.ds(q * CPW + gch * L, L)] = val
                a = a + (nsb[q, pl.ds(ch * L, L)] - val) * npb[q, pl.ds(ch * L, L)]
            accb[pl.ds(gch * L, L)] = a
            cb[pl.ds(gch * L, L)] = jnp.zeros((L,), jnp.float32)

        if h == 0:
            cp_ns2 = pltpu.async_copy(ns_h.at[:, pl.ds(c0 + HC, HC)],
                                      nsb, sems.at[1])
            cp_np2 = pltpu.async_copy(np_h.at[:, pl.ds(c0 + HC, HC)],
                                      npb, sems.at[2])

    def bld_row(b, _):
        ox0 = plsc.load_gather(bndb, [_splat(0), _splat(b)]) * CPW
        ox1 = plsc.load_gather(bndb, [_splat(1), _splat(b)]) * CPW
        oy0 = plsc.load_gather(bndb, [_splat(2), _splat(b)]) * CPW
        oy1 = plsc.load_gather(bndb, [_splat(3), _splat(b)]) * CPW

        @plsc.parallel_loop(0, CPW // L, 1, unroll=4)
        def bld_ch(ch):
            colv = ch * L + iota
            sl = pl.ds(ch * L, L)
            uxb[b, sl] = (plsc.load_gather(u1f, [ox0 + colv])
                          - plsc.load_gather(u1f, [ox1 + colv]))
            uyb[b, sl] = (plsc.load_gather(u1f, [oy0 + colv])
                          - plsc.load_gather(u1f, [oy1 + colv]))
        return 0
    lax.fori_loop(0, NB, bld_row, 0, unroll=False)

    cp_vx.wait()
    cp_vy.wait()
    cp_s1.wait()

    def fp(it, _):
        @plsc.parallel_loop(0, CPW // L, 1, unroll=2)
        def ch_body(ch):
            sl = pl.ds(ch * L, L)
            cv = cb[sl]
            def row(b, carry):
                f, s = carry
                uxv = uxb[b, sl]
                uyv = uyb[b, sl]
                vxv = vxb[b, sl]
                vyv = vyb[b, sl]
                ucx = uxv + cv * vxv
                ucy = uyv + cv * vyv
                u2 = ucx * ucx + ucy * ucy + 1e-4
                ib = plsc.bitcast(u2, jnp.int32)
                y = plsc.bitcast(jnp.int32(0x5F3759DF) - (ib >> 1),
                                 jnp.float32)
                y = y * (1.5 - (0.5 * u2) * y * y)
                f = f + (vxv * vxv + vyv * vyv) * y
                s = s + (uxv * vxv + uyv * vyv) * y
                return f, s
            z = jnp.zeros((L,), jnp.float32)
            f, s = lax.fori_loop(0, NB, row, (z, z), unroll=False)
            cb[sl] = (accb[sl] - ALPHA * s) / (ALPHA * f + s1b[sl])
        return 0
    lax.fori_loop(0, 6, fp, 0, unroll=False)

    cp_dv.wait()
    for rl in range(RPW):

        @plsc.parallel_loop(0, WCW // L, 1, unroll=4)
        def out_ch(ch):
            sl = pl.ds(ch * L, L)
            cg = plsc.load_gather(cb, [cidx[sl]])
            w1b[rl, sl] = w1b[rl, sl] + cg * dvb[rl, sl]

    pltpu.async_copy(w1b, out_h.at[pl.ds(r0, RPW), pl.ds(w0, WCW)],
                     osem).wait()


def _make_sc():
    mesh = plsc.VectorSubcoreMesh(core_axis_name="c", subcore_axis_name="s",
                                  num_cores=2, num_subcores=16)
    return pl.kernel(
        _sc_body,
        out_type=jax.ShapeDtypeStruct((_KS * PR, W1C), jnp.float32),
        mesh=mesh,
        compiler_params=pltpu.CompilerParams(use_tc_tiling_on_sc=False,
                                             needs_layout_passes=False),
        scratch_types=[
            pltpu.VMEM((RPW, WCW), jnp.float32),
            pltpu.VMEM((Q * CPW,), jnp.float32),
            pltpu.VMEM((RPW, WCW), jnp.float32),
            pltpu.VMEM((Q, HC), jnp.float32),
            pltpu.VMEM((Q, HC), jnp.float32),
            pltpu.VMEM((NB, CPW), jnp.float32),
            pltpu.VMEM((NB, CPW), jnp.float32),
            pltpu.VMEM((NB, CPW), jnp.float32),
            pltpu.VMEM((NB, CPW), jnp.float32),
            pltpu.VMEM((CPW,), jnp.float32),
            pltpu.VMEM((CPW,), jnp.float32),
            pltpu.VMEM((CPW,), jnp.float32),
            pltpu.VMEM((4, NBP), jnp.int32),
            pltpu.VMEM((WCW,), jnp.int32),
            pltpu.SemaphoreType.DMA((7,)),
            pltpu.SemaphoreType.DMA,
        ],
    )


BI = 16
NSTEP = TN // BI
OFF = _KS // BI
BC = BI * NC
BR = BI * PR


def _tc_body(w1r, nsr, npr, vxr, vyr, dvr, s1r, sxr, syr, outr, uscr, cscr):
    cidx = lax.broadcasted_iota(jnp.int32, (W1C, NC), 0)
    jidx = lax.broadcasted_iota(jnp.int32, (W1C, NC), 1)
    w1v = w1r[...]
    for g in range(PC):
        csel = (cidx == jidx * PC + g).astype(jnp.float32)
        ag = jnp.dot(w1v, csel, preferred_element_type=jnp.float32)
        for il in range(BI):
            uscr[pl.ds(g * PR, PR), pl.ds(il * NC, NC)] = (
                ag[il * PR:(il + 1) * PR, :])

    uall = uscr[...]
    acc = jnp.sum((nsr[...] - uall) * npr[...], axis=0, keepdims=True)

    ux = jnp.dot(sxr[...], uall, preferred_element_type=jnp.float32)
    uy = jnp.dot(syr[...], uall, preferred_element_type=jnp.float32)

    vxv = vxr[...]
    vyv = vyr[...]
    a1 = vxv * vxv + vyv * vyv
    c1 = ux * vxv + uy * vyv
    s1v = s1r[...].reshape(1, BC)

    c = jnp.zeros((1, BC), jnp.float32)
    for _ in range(6):
        ucx = ux + c * vxv
        ucy = uy + c * vyv
        rb = lax.rsqrt(ucx * ucx + ucy * ucy + 1e-4)
        firs = jnp.sum(a1 * rb, axis=0, keepdims=True)
        sec = jnp.sum(c1 * rb, axis=0, keepdims=True)
        c = (acc - ALPHA * sec) / (ALPHA * firs + s1v)

    for il in range(BI):
        cscr[il, :] = c[0, il * NC:(il + 1) * NC]
    brow = (lax.broadcasted_iota(jnp.int32, (BR, BI), 0) // PR
            == lax.broadcasted_iota(jnp.int32, (BR, BI), 1)
            ).astype(jnp.float32)
    bcol = (lax.broadcasted_iota(jnp.int32, (NC, W1C), 1) // PC
            == lax.broadcasted_iota(jnp.int32, (NC, W1C), 0)
            ).astype(jnp.float32)
    dc = jnp.dot(jnp.dot(brow, cscr[...], preferred_element_type=jnp.float32),
                 bcol, preferred_element_type=jnp.float32)
    outr[...] = w1v + dc * dvr[...]


def _make_tc():
    return pl.pallas_call(
        _tc_body,
        grid=(NSTEP,),
        in_specs=[
            pl.BlockSpec((BR, W1C), lambda i: (OFF + i, 0)),
            pl.BlockSpec((Q, BC), lambda i: (0, OFF + i)),
            pl.BlockSpec((Q, BC), lambda i: (0, OFF + i)),
            pl.BlockSpec((NB, BC), lambda i: (0, OFF + i)),
            pl.BlockSpec((NB, BC), lambda i: (0, OFF + i)),
            pl.BlockSpec((BR, W1C), lambda i: (OFF + i, 0)),
            pl.BlockSpec((BC,), lambda i: (OFF + i,)),
            pl.BlockSpec((NB, Q), lambda i: (0, 0)),
            pl.BlockSpec((NB, Q), lambda i: (0, 0)),
        ],
        out_specs=pl.BlockSpec((BR, W1C), lambda i: (i, 0)),
        out_shape=jax.ShapeDtypeStruct((TN * PR, W1C), jnp.float32),
        scratch_shapes=[pltpu.VMEM((Q, BC), jnp.float32),
                        pltpu.VMEM((BI, NC), jnp.float32)],
        compiler_params=pltpu.CompilerParams(
            dimension_semantics=("arbitrary",),
            vmem_limit_bytes=110 * 1024 * 1024),
    )


def kernel(w1, noise_ch, nphi, vx, vy, dv1, s1, bnd_idx, bnd_idy):
    bnd = jnp.zeros((4, NBP), jnp.int32)
    bnd = bnd.at[0, :NB].set(bnd_idx[:, 0].astype(jnp.int32))
    bnd = bnd.at[1, :NB].set(bnd_idx[:, 1].astype(jnp.int32))
    bnd = bnd.at[2, :NB].set(bnd_idy[:, 0].astype(jnp.int32))
    bnd = bnd.at[3, :NB].set(bnd_idy[:, 1].astype(jnp.int32))

    r0 = _KS * PR
    c0 = _KS * NC
    sc_out = _make_sc()(w1[:r0], noise_ch[:, :c0], nphi[:, :c0],
                        vx[:, :c0], vy[:, :c0], dv1[:r0], s1[:c0], bnd)
    qio = jnp.arange(Q, dtype=jnp.int32)[None, :]
    sx = ((qio == bnd[0, :NB, None]).astype(jnp.float32)
          - (qio == bnd[1, :NB, None]).astype(jnp.float32))
    sy = ((qio == bnd[2, :NB, None]).astype(jnp.float32)
          - (qio == bnd[3, :NB, None]).astype(jnp.float32))
    tc_out = _make_tc()(w1, noise_ch, nphi, vx, vy, dv1, s1, sx, sy)
    return jnp.concatenate([sc_out, tc_out], axis=0)


# speedup vs baseline: 4.6226x; 1.1196x over previous
import functools

import jax
import jax.numpy as jnp
from jax import lax
from jax.experimental import pallas as pl
from jax.experimental.pallas import tpu as pltpu
from jax.experimental.pallas import tpu_sc as plsc

PR, PC = 5, 5
NR, NC = 128, 128
NB = 40
ALPHA = 15.0
Q = PR * PC
W1C = NC * PC
NBP = 48

_KS = 16

L = 16
NWORK = 32
CPW = NC // 2
RPW = PR
WCW = CPW * PC
HC = CPW // 2

TN = NR - _KS
TROWS = TN * PR


def _splat(v):
    return jnp.full((L,), v, jnp.int32)


def _sc_body(wd_h, big_h, bnd_h, out_h,
             w1b, u1f, dvb, nsb, npb, vxb, vyb, uxb, uyb,
             s1b, accb, cb, bndb, cidx, sems, osem):
    wid = lax.axis_index("s") * 2 + lax.axis_index("c")
    iota = lax.iota(jnp.int32, L)

    pltpu.sync_copy(bnd_h, bndb)

    ii = wid // 2
    jh = wid % 2
    r0 = ii * PR
    c0 = ii * NC + jh * CPW
    w0 = jh * WCW

    cp_w1 = pltpu.async_copy(wd_h.at[pl.ds(r0, RPW), pl.ds(w0, WCW)],
                             w1b, sems.at[0])
    cp_ns = pltpu.async_copy(big_h.at[pl.ds(0, Q), pl.ds(c0, HC)],
                             nsb, sems.at[1])
    cp_np = pltpu.async_copy(big_h.at[pl.ds(Q, Q), pl.ds(c0, HC)],
                             npb, sems.at[2])
    cp_vx = pltpu.async_copy(big_h.at[pl.ds(2 * Q, NB), pl.ds(c0, CPW)],
                             vxb, sems.at[3])
    cp_vy = pltpu.async_copy(big_h.at[pl.ds(2 * Q + NB, NB), pl.ds(c0, CPW)],
                             vyb, sems.at[4])
    cp_s1 = pltpu.async_copy(big_h.at[pl.ds(2 * Q + 2 * NB, 1),
                                      pl.ds(c0, CPW)], s1b, sems.at[5])
    cp_dv = pltpu.async_copy(wd_h.at[pl.ds(_KS * PR + r0, RPW),
                                     pl.ds(w0, WCW)], dvb, sems.at[6])

    @plsc.parallel_loop(0, WCW // L, 1, unroll=2)
    def cidx_ch(ch):
        cidx[pl.ds(ch * L, L)] = (ch * L + iota) // PC

    cp_w1.wait()
    for h in range(2):
        if h == 0:
            cp_ns.wait()
            cp_np.wait()
        else:
            cp_ns2.wait()
            cp_np2.wait()

        @plsc.parallel_loop(0, HC // L, 1, unroll=2)
        def build_ch(ch):
            gch = h * (HC // L) + ch
            jjv = gch * L + iota
            a = jnp.zeros((L,), jnp.float32)
            for q in range(Q):
                rowv = _splat(q % PR)
                colv = jjv * PC + (q // PR)
                val = plsc.load_gather(w1b, [rowv, colv])
                u1f[pl.ds(q * CPW + gch * L, L)] = val
                a = a + (nsb[q, pl.ds(ch * L, L)] - val) * npb[q, pl.ds(ch * L, L)]
            accb[pl.ds(gch * L, L)] = a
            cb[pl.ds(gch * L, L)] = jnp.zeros((L,), jnp.float32)

        if h == 0:
            cp_ns2 = pltpu.async_copy(big_h.at[pl.ds(0, Q),
                                               pl.ds(c0 + HC, HC)],
                                      nsb, sems.at[1])
            cp_np2 = pltpu.async_copy(big_h.at[pl.ds(Q, Q),
                                               pl.ds(c0 + HC, HC)],
                                      npb, sems.at[2])

    def bld_row(b, _):
        ox0 = plsc.load_gather(bndb, [_splat(0), _splat(b)]) * CPW
        ox1 = plsc.load_gather(bndb, [_splat(1), _splat(b)]) * CPW
        oy0 = plsc.load_gather(bndb, [_splat(2), _splat(b)]) * CPW
        oy1 = plsc.load_gather(bndb, [_splat(3), _splat(b)]) * CPW

        @plsc.parallel_loop(0, CPW // L, 1, unroll=4)
        def bld_ch(ch):
            colv = ch * L + iota
            sl = pl.ds(ch * L, L)
            uxb[b, sl] = (plsc.load_gather(u1f, [ox0 + colv])
                          - plsc.load_gather(u1f, [ox1 + colv]))
            uyb[b, sl] = (plsc.load_gather(u1f, [oy0 + colv])
                          - plsc.load_gather(u1f, [oy1 + colv]))
        return 0
    lax.fori_loop(0, NB, bld_row, 0, unroll=False)

    cp_vx.wait()
    cp_vy.wait()
    cp_s1.wait()

    def fp(it, _):
        @plsc.parallel_loop(0, CPW // L, 1, unroll=2)
        def ch_body(ch):
            sl = pl.ds(ch * L, L)
            cv = cb[sl]
            def row(b, carry):
                f, s = carry
                uxv = uxb[b, sl]
                uyv = uyb[b, sl]
                vxv = vxb[b, sl]
                vyv = vyb[b, sl]
                ucx = uxv + cv * vxv
                ucy = uyv + cv * vyv
                u2 = ucx * ucx + ucy * ucy + 1e-4
                ib = plsc.bitcast(u2, jnp.int32)
                y = plsc.bitcast(jnp.int32(0x5F3759DF) - (ib >> 1),
                                 jnp.float32)
                y = y * (1.5 - (0.5 * u2) * y * y)
                f = f + (vxv * vxv + vyv * vyv) * y
                s = s + (uxv * vxv + uyv * vyv) * y
                return f, s
            z = jnp.zeros((L,), jnp.float32)
            f, s = lax.fori_loop(0, NB, row, (z, z), unroll=False)
            cb[sl] = (accb[sl] - ALPHA * s) / (ALPHA * f + s1b[0, sl])
        return 0
    lax.fori_loop(0, 6, fp, 0, unroll=False)

    cp_dv.wait()
    for rl in range(RPW):

        @plsc.parallel_loop(0, WCW // L, 1, unroll=4)
        def out_ch(ch):
            sl = pl.ds(ch * L, L)
            cg = plsc.load_gather(cb, [cidx[sl]])
            w1b[rl, sl] = w1b[rl, sl] + cg * dvb[rl, sl]

    pltpu.async_copy(w1b, out_h.at[pl.ds(r0, RPW), pl.ds(w0, WCW)],
                     osem).wait()


def _make_sc():
    mesh = plsc.VectorSubcoreMesh(core_axis_name="c", subcore_axis_name="s",
                                  num_cores=2, num_subcores=16)
    return pl.kernel(
        _sc_body,
        out_type=jax.ShapeDtypeStruct((_KS * PR, W1C), jnp.float32),
        mesh=mesh,
        compiler_params=pltpu.CompilerParams(use_tc_tiling_on_sc=False,
                                             needs_layout_passes=False),
        scratch_types=[
            pltpu.VMEM((RPW, WCW), jnp.float32),
            pltpu.VMEM((Q * CPW,), jnp.float32),
            pltpu.VMEM((RPW, WCW), jnp.float32),
            pltpu.VMEM((Q, HC), jnp.float32),
            pltpu.VMEM((Q, HC), jnp.float32),
            pltpu.VMEM((NB, CPW), jnp.float32),
            pltpu.VMEM((NB, CPW), jnp.float32),
            pltpu.VMEM((NB, CPW), jnp.float32),
            pltpu.VMEM((NB, CPW), jnp.float32),
            pltpu.VMEM((1, CPW), jnp.float32),
            pltpu.VMEM((CPW,), jnp.float32),
            pltpu.VMEM((CPW,), jnp.float32),
            pltpu.VMEM((4, NBP), jnp.int32),
            pltpu.VMEM((WCW,), jnp.int32),
            pltpu.SemaphoreType.DMA((7,)),
            pltpu.SemaphoreType.DMA,
        ],
    )


BI = 16
NSTEP = TN // BI
OFF = _KS // BI
BC = BI * NC
BR = BI * PR


def _tc_body(w1r, nsr, npr, vxr, vyr, dvr, s1r, sxr, syr, outr, uscr, cscr):
    cidx = lax.broadcasted_iota(jnp.int32, (W1C, NC), 0)
    jidx = lax.broadcasted_iota(jnp.int32, (W1C, NC), 1)
    w1v = w1r[...]
    for g in range(PC):
        csel = (cidx == jidx * PC + g).astype(jnp.float32)
        ag = jnp.dot(w1v, csel, preferred_element_type=jnp.float32)
        for il in range(BI):
            uscr[pl.ds(g * PR, PR), pl.ds(il * NC, NC)] = (
                ag[il * PR:(il + 1) * PR, :])

    uall = uscr[...]
    acc = jnp.sum((nsr[...] - uall) * npr[...], axis=0, keepdims=True)

    ux = jnp.dot(sxr[...], uall, preferred_element_type=jnp.float32)
    uy = jnp.dot(syr[...], uall, preferred_element_type=jnp.float32)

    vxv = vxr[...]
    vyv = vyr[...]
    a1 = vxv * vxv + vyv * vyv
    c1 = ux * vxv + uy * vyv
    s1v = s1r[...].reshape(1, BC)

    c = jnp.zeros((1, BC), jnp.float32)
    for _ in range(6):
        ucx = ux + c * vxv
        ucy = uy + c * vyv
        rb = lax.rsqrt(ucx * ucx + ucy * ucy + 1e-4)
        firs = jnp.sum(a1 * rb, axis=0, keepdims=True)
        sec = jnp.sum(c1 * rb, axis=0, keepdims=True)
        c = (acc - ALPHA * sec) / (ALPHA * firs + s1v)

    for il in range(BI):
        cscr[il, :] = c[0, il * NC:(il + 1) * NC]
    brow = (lax.broadcasted_iota(jnp.int32, (BR, BI), 0) // PR
            == lax.broadcasted_iota(jnp.int32, (BR, BI), 1)
            ).astype(jnp.float32)
    bcol = (lax.broadcasted_iota(jnp.int32, (NC, W1C), 1) // PC
            == lax.broadcasted_iota(jnp.int32, (NC, W1C), 0)
            ).astype(jnp.float32)
    dc = jnp.dot(jnp.dot(brow, cscr[...], preferred_element_type=jnp.float32),
                 bcol, preferred_element_type=jnp.float32)
    outr[...] = w1v + dc * dvr[...]


def _make_tc():
    return pl.pallas_call(
        _tc_body,
        grid=(NSTEP,),
        in_specs=[
            pl.BlockSpec((BR, W1C), lambda i: (OFF + i, 0)),
            pl.BlockSpec((Q, BC), lambda i: (0, OFF + i)),
            pl.BlockSpec((Q, BC), lambda i: (0, OFF + i)),
            pl.BlockSpec((NB, BC), lambda i: (0, OFF + i)),
            pl.BlockSpec((NB, BC), lambda i: (0, OFF + i)),
            pl.BlockSpec((BR, W1C), lambda i: (OFF + i, 0)),
            pl.BlockSpec((BC,), lambda i: (OFF + i,)),
            pl.BlockSpec((NB, Q), lambda i: (0, 0)),
            pl.BlockSpec((NB, Q), lambda i: (0, 0)),
        ],
        out_specs=pl.BlockSpec((BR, W1C), lambda i: (i, 0)),
        out_shape=jax.ShapeDtypeStruct((TN * PR, W1C), jnp.float32),
        scratch_shapes=[pltpu.VMEM((Q, BC), jnp.float32),
                        pltpu.VMEM((BI, NC), jnp.float32)],
        compiler_params=pltpu.CompilerParams(
            dimension_semantics=("arbitrary",),
            vmem_limit_bytes=110 * 1024 * 1024),
    )


def kernel(w1, noise_ch, nphi, vx, vy, dv1, s1, bnd_idx, bnd_idy):
    bnd = jnp.pad(jnp.concatenate([bnd_idx.astype(jnp.int32).T,
                                   bnd_idy.astype(jnp.int32).T], axis=0),
                  ((0, 0), (0, NBP - NB)))

    r0 = _KS * PR
    c0 = _KS * NC
    wd = jnp.concatenate([w1[:r0], dv1[:r0]], axis=0)
    big = jnp.concatenate([noise_ch[:, :c0], nphi[:, :c0], vx[:, :c0],
                           vy[:, :c0], s1[None, :c0]], axis=0)
    sc_out = _make_sc()(wd, big, bnd)
    qio = jnp.arange(Q, dtype=jnp.int32)[None, :]
    sx = ((qio == bnd[0, :NB, None]).astype(jnp.float32)
          - (qio == bnd[1, :NB, None]).astype(jnp.float32))
    sy = ((qio == bnd[2, :NB, None]).astype(jnp.float32)
          - (qio == bnd[3, :NB, None]).astype(jnp.float32))
    tc_out = _make_tc()(w1, noise_ch, nphi, vx, vy, dv1, s1, sx, sy)
    return jnp.concatenate([sc_out, tc_out], axis=0)
